# baseline (speedup 1.0000x reference)
import functools

import jax
import jax.numpy as jnp
from jax import lax
from jax.experimental import pallas as pl
from jax.experimental.pallas import tpu as pltpu
from jax.experimental.pallas import tpu_sc as plsc

N = 16384
C = 1000
CSPLIT = 896
CT = C - CSPLIT
NW = 32
RPW = N // NW
CH = 64
NCH = RPW // CH
NBUF = 3
TFULL = (CT // 16) * 16
TTAIL = CT - TFULL
BR = 512
GRID = N // BR


def _sc_tail_body(idx_hbm, out_hbm, idx_v, bufs, sem):
    wid = lax.axis_index("s") * 2 + lax.axis_index("c")
    base_row = wid * RPW
    pltpu.sync_copy(idx_hbm.at[pl.ds(base_row, RPW)], idx_v)

    lanes = lax.iota(jnp.int32, 16)
    zeros = jnp.zeros((16,), jnp.float32)
    ones = jnp.ones((16,), jnp.float32)
    tail_mask = lanes < TTAIL

    for buf in bufs:
        def _zero_row(r, _, buf=buf):
            for cc in range(TFULL // 16):
                buf[r, pl.ds(cc * 16, 16)] = zeros
            rows = jnp.full((16,), r, jnp.int32)
            plsc.store_scatter(buf, [rows, TFULL + lanes], zeros,
                               mask=tail_mask)
            return _
        lax.fori_loop(0, CH, _zero_row, 0)

    copies = [None] * NCH
    for c in range(NCH):
        b = bufs[c % NBUF]
        if c >= NBUF:
            copies[c - NBUF].wait()
            for g in range(CH // 16):
                oldcols = idx_v[pl.ds((c - NBUF) * CH + g * 16, 16)]
                plsc.store_scatter(b, [lanes + g * 16, oldcols - CSPLIT],
                                   zeros, mask=oldcols >= CSPLIT)
        for g in range(CH // 16):
            cols = idx_v[pl.ds(c * CH + g * 16, 16)]
            plsc.store_scatter(b, [lanes + g * 16, cols - CSPLIT], ones,
                               mask=cols >= CSPLIT)
        copies[c] = pltpu.async_copy(
            b,
            out_hbm.at[pl.ds(base_row + c * CH, CH), pl.ds(CSPLIT, CT)],
            sem.at[c % NBUF])
    for c in range(NCH - NBUF, NCH):
        copies[c].wait()


def _tc_main_body(idx_ref, alias_ref, out_ref):
    del alias_ref
    idx = idx_ref[0, 0, :]
    cols = jax.lax.broadcasted_iota(jnp.int32, (BR, CSPLIT), 1)
    out_ref[...] = jnp.where(idx[:, None] == cols, jnp.float32(1.0),
                             jnp.float32(0.0))


def kernel(inputs):
    idx = inputs.astype(jnp.int32)

    mesh = plsc.VectorSubcoreMesh(core_axis_name="c", subcore_axis_name="s")
    sc_run = functools.partial(
        pl.kernel,
        mesh=mesh,
        out_type=jax.ShapeDtypeStruct((N, C), jnp.float32),
        scratch_types=[
            pltpu.VMEM((RPW,), jnp.int32),
            tuple(pltpu.VMEM((CH, CT), jnp.float32) for _ in range(NBUF)),
            pltpu.SemaphoreType.DMA((NBUF,)),
        ],
        compiler_params=pltpu.CompilerParams(needs_layout_passes=False),
    )(_sc_tail_body)
    partial_out = sc_run(idx)

    idx3 = idx.reshape(GRID, 1, BR)
    out = pl.pallas_call(
        _tc_main_body,
        grid=(GRID,),
        in_specs=[
            pl.BlockSpec((1, 1, BR), lambda i: (i, 0, 0)),
            pl.BlockSpec(memory_space=pltpu.MemorySpace.HBM),
        ],
        out_specs=pl.BlockSpec((BR, CSPLIT), lambda i: (i, 0)),
        out_shape=jax.ShapeDtypeStruct((N, C), jnp.float32),
        input_output_aliases={1: 0},
    )(idx3, partial_out)
    return out


# speedup vs baseline: 1.0625x; 1.0625x over previous
import functools

import jax
import jax.numpy as jnp
from jax import lax
from jax.experimental import pallas as pl
from jax.experimental.pallas import tpu as pltpu
from jax.experimental.pallas import tpu_sc as plsc

N = 16384
C = 1000
CSPLIT = 896
CT = C - CSPLIT
NW = 32
RPW = N // NW
CH = 64
NCH = RPW // CH
NBUF = 3
TFULL = (CT // 16) * 16
TTAIL = CT - TFULL
BR = 512
GRID = N // BR


def _sc_tail_body(idx_hbm, out_hbm, idx_v, bufs, sem):
    wid = lax.axis_index("s") * 2 + lax.axis_index("c")
    base_row = wid * RPW
    pltpu.sync_copy(idx_hbm.at[pl.ds(base_row, RPW)], idx_v)

    lanes = lax.iota(jnp.int32, 16)
    zeros = jnp.zeros((16,), jnp.float32)
    ones = jnp.ones((16,), jnp.float32)
    tail_mask = lanes < TTAIL

    for buf in bufs:
        def _zero_row(r, _, buf=buf):
            for cc in range(TFULL // 16):
                buf[r, pl.ds(cc * 16, 16)] = zeros
            rows = jnp.full((16,), r, jnp.int32)
            plsc.store_scatter(buf, [rows, TFULL + lanes], zeros,
                               mask=tail_mask)
            return _
        lax.fori_loop(0, CH, _zero_row, 0)

    copies = [None] * NCH
    for c in range(NCH):
        b = bufs[c % NBUF]
        if c >= NBUF:
            copies[c - NBUF].wait()
            for g in range(CH // 16):
                oldcols = idx_v[pl.ds((c - NBUF) * CH + g * 16, 16)]
                plsc.store_scatter(b, [lanes + g * 16, oldcols - CSPLIT],
                                   zeros, mask=oldcols >= CSPLIT)
        for g in range(CH // 16):
            cols = idx_v[pl.ds(c * CH + g * 16, 16)]
            plsc.store_scatter(b, [lanes + g * 16, cols - CSPLIT], ones,
                               mask=cols >= CSPLIT)
        copies[c] = pltpu.async_copy(
            b,
            out_hbm.at[pl.ds(base_row + c * CH, CH), pl.ds(CSPLIT, CT)],
            sem.at[c % NBUF])
    for c in range(NCH - NBUF, NCH):
        copies[c].wait()


TCBUF = 4


def _tc_main_body(idx_ref, alias_ref, out_ref, buf, sem):
    del alias_ref
    i = pl.program_id(0)
    slot = lax.rem(i, TCBUF)

    @pl.when(i >= TCBUF)
    def _():
        pltpu.make_async_copy(
            buf.at[slot], out_ref.at[pl.ds(0, BR), pl.ds(0, CSPLIT)],
            sem.at[slot]).wait()

    idx = idx_ref[0, 0, :]
    cols = jax.lax.broadcasted_iota(jnp.int32, (BR, CSPLIT), 1)
    buf[slot] = jnp.where(idx[:, None] == cols, jnp.float32(1.0),
                          jnp.float32(0.0))
    pltpu.make_async_copy(
        buf.at[slot], out_ref.at[pl.ds(i * BR, BR), pl.ds(0, CSPLIT)],
        sem.at[slot]).start()

    @pl.when(i == GRID - 1)
    def _():
        for s in range(TCBUF):
            pltpu.make_async_copy(
                buf.at[s], out_ref.at[pl.ds(0, BR), pl.ds(0, CSPLIT)],
                sem.at[s]).wait()


def kernel(inputs):
    idx = inputs.astype(jnp.int32)

    mesh = plsc.VectorSubcoreMesh(core_axis_name="c", subcore_axis_name="s")
    sc_run = functools.partial(
        pl.kernel,
        mesh=mesh,
        out_type=jax.ShapeDtypeStruct((N, C), jnp.float32),
        scratch_types=[
            pltpu.VMEM((RPW,), jnp.int32),
            tuple(pltpu.VMEM((CH, CT), jnp.float32) for _ in range(NBUF)),
            pltpu.SemaphoreType.DMA((NBUF,)),
        ],
        compiler_params=pltpu.CompilerParams(needs_layout_passes=False),
    )(_sc_tail_body)
    partial_out = sc_run(idx)

    idx3 = idx.reshape(GRID, 1, BR)
    out = pl.pallas_call(
        _tc_main_body,
        grid=(GRID,),
        in_specs=[
            pl.BlockSpec((1, 1, BR), lambda i: (i, 0, 0)),
            pl.BlockSpec(memory_space=pltpu.MemorySpace.HBM),
        ],
        out_specs=pl.BlockSpec(memory_space=pltpu.MemorySpace.HBM),
        out_shape=jax.ShapeDtypeStruct((N, C), jnp.float32),
        input_output_aliases={1: 0},
        scratch_shapes=[
            pltpu.VMEM((TCBUF, BR, CSPLIT), jnp.float32),
            pltpu.SemaphoreType.DMA((TCBUF,)),
        ],
    )(idx3, partial_out)
    return out


# speedup vs baseline: 1.0627x; 1.0002x over previous
import functools

import jax
import jax.numpy as jnp
from jax import lax
from jax.experimental import pallas as pl
from jax.experimental.pallas import tpu as pltpu
from jax.experimental.pallas import tpu_sc as plsc

N = 16384
C = 1000
CSPLIT = 896
CT = C - CSPLIT
NW = 32
RPW = N // NW
CHA = 32
NCHA = RPW // CHA
CHB = 64
NCHB = RPW // CHB
TFULL = (CT // 16) * 16
TTAIL = CT - TFULL


def _sc_body(idx_hbm, out_hbm, idx_v, bufa0, bufa1, bufb0, bufb1, sem):
    wid = lax.axis_index("s") * 2 + lax.axis_index("c")
    base_row = wid * RPW
    pltpu.sync_copy(idx_hbm.at[pl.ds(base_row, RPW)], idx_v)

    lanes = lax.iota(jnp.int32, 16)
    zeros = jnp.zeros((16,), jnp.float32)
    ones = jnp.ones((16,), jnp.float32)

    for buf in (bufa0, bufa1):
        def _zero_row_a(r, _, buf=buf):
            for cc in range(CSPLIT // 16):
                buf[r, pl.ds(cc * 16, 16)] = zeros
            return _
        lax.fori_loop(0, CHA, _zero_row_a, 0)
    for buf in (bufb0, bufb1):
        def _zero_row_b(r, _, buf=buf):
            for cc in range(TFULL // 16):
                buf[r, pl.ds(cc * 16, 16)] = zeros
            rows = jnp.full((16,), r, jnp.int32)
            plsc.store_scatter(buf, [rows, TFULL + lanes], zeros,
                               mask=lanes < TTAIL)
            return _
        lax.fori_loop(0, CHB, _zero_row_b, 0)

    bufsa = (bufa0, bufa1)
    copies = [None] * NCHA
    for c in range(NCHA):
        b = bufsa[c % 2]
        if c >= 2:
            copies[c - 2].wait()
            for g in range(CHA // 16):
                oldcols = idx_v[pl.ds((c - 2) * CHA + g * 16, 16)]
                plsc.store_scatter(b, [lanes + g * 16, oldcols], zeros,
                                   mask=oldcols < CSPLIT)
        for g in range(CHA // 16):
            cols = idx_v[pl.ds(c * CHA + g * 16, 16)]
            plsc.store_scatter(b, [lanes + g * 16, cols], ones,
                               mask=cols < CSPLIT)
        copies[c] = pltpu.async_copy(
            b, out_hbm.at[pl.ds(base_row + c * CHA, CHA), pl.ds(0, CSPLIT)],
            sem.at[c % 2])
    for c in range(NCHA - 2, NCHA):
        copies[c].wait()

    bufsb = (bufb0, bufb1)
    copies = [None] * NCHB
    for c in range(NCHB):
        b = bufsb[c % 2]
        if c >= 2:
            copies[c - 2].wait()
            for g in range(CHB // 16):
                oldcols = idx_v[pl.ds((c - 2) * CHB + g * 16, 16)]
                plsc.store_scatter(b, [lanes + g * 16, oldcols - CSPLIT],
                                   zeros, mask=oldcols >= CSPLIT)
        for g in range(CHB // 16):
            cols = idx_v[pl.ds(c * CHB + g * 16, 16)]
            plsc.store_scatter(b, [lanes + g * 16, cols - CSPLIT], ones,
                               mask=cols >= CSPLIT)
        copies[c] = pltpu.async_copy(
            b, out_hbm.at[pl.ds(base_row + c * CHB, CHB), pl.ds(CSPLIT, CT)],
            sem.at[2 + c % 2])
    for c in range(NCHB - 2, NCHB):
        copies[c].wait()


def kernel(inputs):
    idx = inputs.astype(jnp.int32)
    mesh = plsc.VectorSubcoreMesh(core_axis_name="c", subcore_axis_name="s")
    run = functools.partial(
        pl.kernel,
        mesh=mesh,
        out_type=jax.ShapeDtypeStruct((N, C), jnp.float32),
        scratch_types=[
            pltpu.VMEM((RPW,), jnp.int32),
            pltpu.VMEM((CHA, CSPLIT), jnp.float32),
            pltpu.VMEM((CHA, CSPLIT), jnp.float32),
            pltpu.VMEM((CHB, CT), jnp.float32),
            pltpu.VMEM((CHB, CT), jnp.float32),
            pltpu.SemaphoreType.DMA((4,)),
        ],
        compiler_params=pltpu.CompilerParams(needs_layout_passes=False),
    )(_sc_body)
    return run(idx)


# speedup vs baseline: 1.0874x; 1.0233x over previous
import functools

import jax
import jax.numpy as jnp
from jax import lax
from jax.experimental import pallas as pl
from jax.experimental.pallas import tpu as pltpu
from jax.experimental.pallas import tpu_sc as plsc

N = 16384
C = 1000
NW = 32
RPW = N // NW
CH = 16
NCH = RPW // CH
CFULL = (C // 16) * 16
CTAIL = C - CFULL


def _sc_body(idx_hbm, out_hbm, idx_v, buf0, buf1, sem):
    wid = lax.axis_index("s") * 2 + lax.axis_index("c")
    base_row = wid * RPW
    pltpu.sync_copy(idx_hbm.at[pl.ds(base_row, RPW)], idx_v)

    lanes = lax.iota(jnp.int32, 16)
    zeros = jnp.zeros((16,), jnp.float32)
    ones = jnp.ones((16,), jnp.float32)
    tail_mask = lanes < CTAIL

    for buf in (buf0, buf1):
        def _zero_row(r, _, buf=buf):
            for cc in range(CFULL // 16):
                buf[r, pl.ds(cc * 16, 16)] = zeros
            rows = jnp.full((16,), r, jnp.int32)
            plsc.store_scatter(buf, [rows, CFULL + lanes], zeros,
                               mask=tail_mask)
            return _
        lax.fori_loop(0, CH, _zero_row, 0)

    bufs = (buf0, buf1)
    copies = [None] * NCH
    for c in range(NCH):
        b = bufs[c % 2]
        cols = idx_v[pl.ds(c * CH, 16)]
        if c >= 2:
            copies[c - 2].wait()
            oldcols = idx_v[pl.ds((c - 2) * CH, 16)]
            plsc.store_scatter(b, [lanes, oldcols], zeros)
        plsc.store_scatter(b, [lanes, cols], ones)
        copies[c] = pltpu.async_copy(
            b, out_hbm.at[pl.ds(base_row + c * CH, CH)], sem.at[c % 2])
    copies[NCH - 2].wait()
    copies[NCH - 1].wait()


def kernel(inputs):
    idx = inputs.astype(jnp.int32)
    mesh = plsc.VectorSubcoreMesh(core_axis_name="c", subcore_axis_name="s")
    run = functools.partial(
        pl.kernel,
        mesh=mesh,
        out_type=jax.ShapeDtypeStruct((N, C), jnp.float32),
        scratch_types=[
            pltpu.VMEM((RPW,), jnp.int32),
            pltpu.VMEM((CH, C), jnp.float32),
            pltpu.VMEM((CH, C), jnp.float32),
            pltpu.SemaphoreType.DMA((2,)),
        ],
        compiler_params=pltpu.CompilerParams(needs_layout_passes=False),
    )(_sc_body)
    return run(idx)
